# baseline (speedup 1.0000x reference)
import functools
import math

import jax
import jax.numpy as jnp
from jax import lax
from jax.experimental import pallas as pl
from jax.experimental.pallas import tpu as pltpu
from jax.experimental.pallas import tpu_sc as plsc

BLK = 2048
K = 100


def _history_gather(item_table, flat_idx):
    bh = flat_idx.shape[0]
    d = item_table.shape[1]
    try:
        info = plsc.get_sparse_core_info()
    except ValueError:
        return jnp.take(item_table, flat_idx, axis=0)
    nw = info.num_cores * info.num_subcores
    if bh % nw != 0 or (bh // nw) % 8 != 0:
        return jnp.take(item_table, flat_idx, axis=0)
    b_per_w = bh // nw
    chunk = b_per_w // 4 if b_per_w % 4 == 0 and b_per_w >= 32 else b_per_w
    n_chunks = b_per_w // chunk
    mesh = plsc.VectorSubcoreMesh(core_axis_name="c", subcore_axis_name="s")

    @functools.partial(
        pl.kernel, mesh=mesh,
        out_type=jax.ShapeDtypeStruct((bh, d), jnp.float32),
        scratch_types=[
            pltpu.VMEM((chunk,), jnp.int32),
            pltpu.VMEM((chunk, d), jnp.float32),
            pltpu.SemaphoreType.DMA,
        ],
    )
    def gather_k(table_hbm, idx_hbm, out_hbm, idx_v, rows_v, sem):
        wid = lax.axis_index("s") * info.num_cores + lax.axis_index("c")
        base = wid * b_per_w
        for c in range(n_chunks):
            pltpu.sync_copy(idx_hbm.at[pl.ds(base + c * chunk, chunk)], idx_v)
            pltpu.async_copy(table_hbm.at[idx_v], rows_v, sem).wait()
            pltpu.sync_copy(rows_v, out_hbm.at[pl.ds(base + c * chunk, chunk)])

    return gather_k(item_table, flat_idx)


def _scores_body(iv, u_ref, t_ref, s_ref, m_ref):
    i = pl.program_id(0)
    s = lax.dot_general(
        u_ref[...], t_ref[...], (((1,), (1,)), ((), ())),
        preferred_element_type=jnp.float32)
    idx = i * BLK + lax.broadcasted_iota(jnp.int32, s.shape, 1)
    s = jnp.where(idx < iv, s, -jnp.inf)
    s_ref[...] = s
    b = s.shape[0]
    m_ref[0] = s.reshape(b, 4, 4, 128).max(axis=2).reshape(b, 512)


def _scores_and_groupmax(u, item_table):
    b, d = u.shape
    iv = item_table.shape[0]
    n_pad = math.ceil(iv / BLK) * BLK
    if n_pad != iv:
        item_table = jnp.pad(item_table, ((0, n_pad - iv), (0, 0)))
    grid = (n_pad // BLK,)
    return pl.pallas_call(
        functools.partial(_scores_body, iv),
        grid=grid,
        in_specs=[
            pl.BlockSpec((b, d), lambda i: (0, 0)),
            pl.BlockSpec((BLK, d), lambda i: (i, 0)),
        ],
        out_specs=[
            pl.BlockSpec((b, BLK), lambda i: (0, i)),
            pl.BlockSpec((1, b, 512), lambda i: (i, 0, 0)),
        ],
        out_shape=[
            jax.ShapeDtypeStruct((b, n_pad), jnp.float32),
            jax.ShapeDtypeStruct((n_pad // BLK, b, 512), jnp.float32),
        ],
    )(u, item_table)


def _refine(vals, child_ids, k):
    b, n, r = child_ids.shape
    flat = child_ids.reshape(b, n * r)
    cand = jnp.take_along_axis(vals, flat, axis=1)
    _, pos = lax.top_k(cand, min(k, n * r))
    return jnp.take_along_axis(flat, pos, axis=1)


def kernel(user_id, user_features, user_history, user_id_table, item_id_table,
           Wf, bf, Wt, bt):
    bb, hh = user_history.shape
    user_history_embedding = _history_gather(
        item_id_table, user_history.reshape(bb * hh)).reshape(bb, hh, -1)
    user_history_summary = user_history_embedding.mean(axis=1)
    user_id_embedding = jnp.take(user_id_table, user_id, axis=0)
    user_features_embedding = user_features @ Wf.T + bf
    user_tower_input = jnp.concatenate(
        [user_id_embedding, user_features_embedding, user_history_summary],
        axis=1)
    user_embedding = user_tower_input @ Wt.T + bt

    scores, sub3 = _scores_and_groupmax(user_embedding, item_id_table)
    b, n_pad = scores.shape
    nblk = n_pad // BLK

    st = sub3.transpose(1, 0, 2)
    a4 = st.reshape(b, nblk * 512)
    a16 = st.reshape(b, nblk, 4, 128).max(axis=2)
    a64 = a16.reshape(b, nblk, 32, 4).max(axis=3)
    a256 = a64.reshape(b, nblk, 8, 4).max(axis=3)
    a16 = a16.reshape(b, nblk * 128)
    a64 = a64.reshape(b, nblk * 32)
    a256 = a256.reshape(b, nblk * 8)

    d4 = jnp.arange(4, dtype=jnp.int32)
    kz = min(K, nblk * 8)
    _, z = lax.top_k(a256, kz)
    y = _refine(a64, (z // 8 * 32 + z % 8 * 4)[:, :, None] + d4, K)
    g = _refine(a16, (y // 32 * 128 + y % 32 * 4)[:, :, None] + d4, K)
    s4 = _refine(a4, (g // 128 * 512 + g % 128)[:, :, None] + d4 * 128, K)
    i_ = s4 // 512
    m_ = s4 % 512 // 128
    x_ = s4 % 128
    items = (i_ * 2048 + m_ * 512 + x_)[:, :, None] + d4 * 128
    top_items = _refine(scores, items, K)
    return top_items


# speedup vs baseline: 1.1629x; 1.1629x over previous
import functools
import math

import jax
import jax.numpy as jnp
from jax import lax
from jax.experimental import pallas as pl
from jax.experimental.pallas import tpu as pltpu
from jax.experimental.pallas import tpu_sc as plsc

BLK = 2048
K = 100


def _history_gather(item_table, flat_idx):
    bh = flat_idx.shape[0]
    d = item_table.shape[1]
    try:
        info = plsc.get_sparse_core_info()
    except ValueError:
        return jnp.take(item_table, flat_idx, axis=0)
    nw = info.num_cores * info.num_subcores
    if bh % nw != 0 or (bh // nw) % 8 != 0:
        return jnp.take(item_table, flat_idx, axis=0)
    b_per_w = bh // nw
    chunk = b_per_w // 4 if b_per_w % 4 == 0 and b_per_w >= 32 else b_per_w
    n_chunks = b_per_w // chunk
    mesh = plsc.VectorSubcoreMesh(core_axis_name="c", subcore_axis_name="s")

    @functools.partial(
        pl.kernel, mesh=mesh,
        out_type=jax.ShapeDtypeStruct((bh, d), jnp.float32),
        scratch_types=[
            pltpu.VMEM((chunk,), jnp.int32),
            pltpu.VMEM((chunk, d), jnp.float32),
            pltpu.SemaphoreType.DMA,
        ],
    )
    def gather_k(table_hbm, idx_hbm, out_hbm, idx_v, rows_v, sem):
        wid = lax.axis_index("s") * info.num_cores + lax.axis_index("c")
        base = wid * b_per_w
        for c in range(n_chunks):
            pltpu.sync_copy(idx_hbm.at[pl.ds(base + c * chunk, chunk)], idx_v)
            pltpu.async_copy(table_hbm.at[idx_v], rows_v, sem).wait()
            pltpu.sync_copy(rows_v, out_hbm.at[pl.ds(base + c * chunk, chunk)])

    return gather_k(item_table, flat_idx)


def _scores_body(iv, u_ref, t_ref, s_ref, m_ref, g_ref):
    i = pl.program_id(0)
    s = lax.dot_general(
        u_ref[...], t_ref[...], (((1,), (1,)), ((), ())),
        preferred_element_type=jnp.float32)
    idx = i * BLK + lax.broadcasted_iota(jnp.int32, s.shape, 1)
    s = jnp.where(idx < iv, s, -jnp.inf)
    s_ref[...] = s
    b = s.shape[0]
    sub = s.reshape(b, 4, 4, 128).max(axis=2)
    m_ref[...] = sub.reshape(b, 512)
    g_ref[...] = sub.max(axis=1)


def _scores_and_groupmax(u, item_table):
    b, d = u.shape
    iv = item_table.shape[0]
    n_pad = math.ceil(iv / BLK) * BLK
    if n_pad != iv:
        item_table = jnp.pad(item_table, ((0, n_pad - iv), (0, 0)))
    grid = (n_pad // BLK,)
    return pl.pallas_call(
        functools.partial(_scores_body, iv),
        grid=grid,
        in_specs=[
            pl.BlockSpec((b, d), lambda i: (0, 0)),
            pl.BlockSpec((BLK, d), lambda i: (i, 0)),
        ],
        out_specs=[
            pl.BlockSpec((b, BLK), lambda i: (0, i)),
            pl.BlockSpec((b, 512), lambda i: (0, i)),
            pl.BlockSpec((b, 128), lambda i: (0, i)),
        ],
        out_shape=[
            jax.ShapeDtypeStruct((b, n_pad), jnp.float32),
            jax.ShapeDtypeStruct((b, n_pad // 4), jnp.float32),
            jax.ShapeDtypeStruct((b, n_pad // 16), jnp.float32),
        ],
    )(u, item_table)


def _refine(vals, child_ids, k):
    b, n, r = child_ids.shape
    flat = child_ids.reshape(b, n * r)
    cand = jnp.take_along_axis(vals, flat, axis=1)
    _, pos = lax.top_k(cand, min(k, n * r))
    return jnp.take_along_axis(flat, pos, axis=1)


def kernel(user_id, user_features, user_history, user_id_table, item_id_table,
           Wf, bf, Wt, bt):
    bb, hh = user_history.shape
    user_history_embedding = _history_gather(
        item_id_table, user_history.reshape(bb * hh)).reshape(bb, hh, -1)
    user_history_summary = user_history_embedding.mean(axis=1)
    user_id_embedding = jnp.take(user_id_table, user_id, axis=0)
    user_features_embedding = user_features @ Wf.T + bf
    user_tower_input = jnp.concatenate(
        [user_id_embedding, user_features_embedding, user_history_summary],
        axis=1)
    user_embedding = user_tower_input @ Wt.T + bt

    scores, a4, a16 = _scores_and_groupmax(user_embedding, item_id_table)
    b, n_pad = scores.shape
    nblk = n_pad // BLK

    a64 = a16.reshape(b, nblk * 32, 4).max(axis=2)
    a256 = a64.reshape(b, nblk * 8, 4).max(axis=2)

    d4 = jnp.arange(4, dtype=jnp.int32)
    kz = min(K, nblk * 8)
    _, z = lax.top_k(a256, kz)
    y = _refine(a64, (z // 8 * 32 + z % 8 * 4)[:, :, None] + d4, K)
    g = _refine(a16, (y // 32 * 128 + y % 32 * 4)[:, :, None] + d4, K)
    s4 = _refine(a4, (g // 128 * 512 + g % 128)[:, :, None] + d4 * 128, K)
    i_ = s4 // 512
    m_ = s4 % 512 // 128
    x_ = s4 % 128
    items = (i_ * 2048 + m_ * 512 + x_)[:, :, None] + d4 * 128
    top_items = _refine(scores, items, K)
    return top_items
